# speedup vs baseline: 1.6647x; 1.6647x over previous
import jax
import jax.numpy as jnp
from jax.experimental import pallas as pl
from jax.experimental.pallas import tpu as pltpu

_E = 8
_K = 2
_TB = 512


def _moe_body(x_ref, sgT_ref, suT_ref, sdT_ref, rT_ref,
              egT_ref, euT_ref, edT_ref, out_ref, acc_ref, comb_ref):
    e = pl.program_id(1)
    x = x_ref[...]

    @pl.when(e == 0)
    def _first():
        g = jnp.dot(x, sgT_ref[...], preferred_element_type=jnp.float32)
        u = jnp.dot(x, suT_ref[...], preferred_element_type=jnp.float32)
        h = jax.nn.sigmoid(g) * u
        sh = jnp.dot(h, sdT_ref[...], preferred_element_type=jnp.float32)
        logits = jnp.dot(x, rT_ref[...], preferred_element_type=jnp.float32)
        m = jnp.max(logits, axis=-1, keepdims=True)
        ex = jnp.exp(logits - m)
        sm = ex / jnp.sum(ex, axis=-1, keepdims=True)
        ids = jax.lax.broadcasted_iota(jnp.int32, sm.shape, 1)
        m1 = jnp.max(sm, axis=-1, keepdims=True)
        i1 = jnp.min(jnp.where(sm == m1, ids, _E), axis=-1, keepdims=True)
        s2 = jnp.where(ids == i1, -jnp.inf, sm)
        m2 = jnp.max(s2, axis=-1, keepdims=True)
        i2 = jnp.min(jnp.where(s2 == m2, ids, _E), axis=-1, keepdims=True)
        comb_ref[...] = (jnp.where(ids == i1, m1, 0.0)
                         + jnp.where(ids == i2, m2, 0.0))
        acc_ref[...] = x + sh

    ge = jnp.dot(x, egT_ref[0], preferred_element_type=jnp.float32)
    ue = jnp.dot(x, euT_ref[0], preferred_element_type=jnp.float32)
    he = jax.nn.sigmoid(ge) * ue
    ye = jnp.dot(he, edT_ref[0], preferred_element_type=jnp.float32)
    ids = jax.lax.broadcasted_iota(jnp.int32, comb_ref.shape, 1)
    ce = jnp.sum(jnp.where(ids == e, comb_ref[...], 0.0), axis=-1,
                 keepdims=True)
    acc_ref[...] += ce * ye

    @pl.when(e == _E - 1)
    def _last():
        out_ref[...] = acc_ref[...]


def kernel(hidden_states, shared_gate_w, shared_up_w, shared_down_w,
           expert_gate_w, expert_up_w, expert_down_w, router_w):
    B, S, H = hidden_states.shape
    I = shared_gate_w.shape[0]
    T = B * S
    x2 = hidden_states.reshape(T, H)
    sgT = shared_gate_w.T
    suT = shared_up_w.T
    sdT = shared_down_w.T
    rT = router_w.T
    egT = expert_gate_w.transpose(0, 2, 1)
    euT = expert_up_w.transpose(0, 2, 1)
    edT = expert_down_w.transpose(0, 2, 1)

    nt = T // _TB
    out = pl.pallas_call(
        _moe_body,
        grid=(nt, _E),
        in_specs=[
            pl.BlockSpec((_TB, H), lambda t, e: (t, 0)),
            pl.BlockSpec((H, I), lambda t, e: (0, 0)),
            pl.BlockSpec((H, I), lambda t, e: (0, 0)),
            pl.BlockSpec((I, H), lambda t, e: (0, 0)),
            pl.BlockSpec((H, _E), lambda t, e: (0, 0)),
            pl.BlockSpec((1, H, I), lambda t, e: (e, 0, 0)),
            pl.BlockSpec((1, H, I), lambda t, e: (e, 0, 0)),
            pl.BlockSpec((1, I, H), lambda t, e: (e, 0, 0)),
        ],
        out_specs=pl.BlockSpec((_TB, H), lambda t, e: (t, 0)),
        out_shape=jax.ShapeDtypeStruct((T, H), jnp.float32),
        scratch_shapes=[
            pltpu.VMEM((_TB, H), jnp.float32),
            pltpu.VMEM((_TB, _E), jnp.float32),
        ],
    )(x2, sgT, suT, sdT, rT, egT, euT, edT)
    return out.reshape(B, S, H)


# speedup vs baseline: 1.7066x; 1.0252x over previous
import functools

import jax
import jax.numpy as jnp
from jax import lax
from jax.experimental import pallas as pl
from jax.experimental.pallas import tpu as pltpu
from jax.experimental.pallas import tpu_sc as plsc

_E = 8
_K = 2
_TB = 512
_BLK = 256
_NTILES = 32
_L = 16


def _router_body(x_ref, sgT_ref, suT_ref, sdT_ref, rT_ref,
                 xs_ref, meta_ref, cnt_ref, carry_ref):
    t = pl.program_id(0)

    @pl.when(t == 0)
    def _init():
        carry_ref[...] = jnp.zeros_like(carry_ref)

    x = x_ref[...]
    g = jnp.dot(x, sgT_ref[...], preferred_element_type=jnp.float32)
    u = jnp.dot(x, suT_ref[...], preferred_element_type=jnp.float32)
    h = jax.nn.sigmoid(g) * u
    sh = jnp.dot(h, sdT_ref[...], preferred_element_type=jnp.float32)
    xs_ref[...] = x + sh

    logits = jnp.dot(x, rT_ref[...], preferred_element_type=jnp.float32)
    m = jnp.max(logits, axis=-1, keepdims=True)
    ex = jnp.exp(logits - m)
    sm = ex / jnp.sum(ex, axis=-1, keepdims=True)
    ids = jax.lax.broadcasted_iota(jnp.int32, sm.shape, 1)
    m1 = jnp.max(sm, axis=-1, keepdims=True)
    i1 = jnp.min(jnp.where(sm == m1, ids, _E), axis=-1, keepdims=True)
    s2 = jnp.where(ids == i1, -jnp.inf, sm)
    m2 = jnp.max(s2, axis=-1, keepdims=True)
    i2 = jnp.min(jnp.where(s2 == m2, ids, _E), axis=-1, keepdims=True)

    oh0 = (ids == i1).astype(jnp.float32)
    oh1 = (ids == i2).astype(jnp.float32)
    oh = oh0 + oh1
    row = jax.lax.broadcasted_iota(jnp.int32, (_TB, _TB), 0)
    col = jax.lax.broadcasted_iota(jnp.int32, (_TB, _TB), 1)
    tril = (row > col).astype(jnp.float32)
    c = jnp.dot(tril, oh, preferred_element_type=jnp.float32) + carry_ref[...]
    r0 = jnp.sum(c * oh0, axis=-1, keepdims=True)
    r1 = jnp.sum(c * oh1, axis=-1, keepdims=True)
    carry_new = carry_ref[...] + jnp.sum(oh, axis=0, keepdims=True)
    carry_ref[...] = carry_new
    cnt_ref[...] = jnp.broadcast_to(carry_new, (8, 8))

    lane = jax.lax.broadcasted_iota(jnp.int32, (_TB, 8), 1)
    meta = jnp.where(
        lane == 0, m1,
        jnp.where(lane == 1, m2,
                  jnp.where(lane == 2, i1.astype(jnp.float32),
                            jnp.where(lane == 3, i2.astype(jnp.float32),
                                      jnp.where(lane == 4, r0,
                                                jnp.where(lane == 5, r1,
                                                          0.0))))))
    meta_ref[...] = meta[None]


def _scatter_body(x2, i0, i1, r0, r1, ps, xg, sp0, sp1,
                  iv0, iv1, rv0, rv1, psv, spf0, spf1, sp2d0, sp2d1,
                  xbuf, sem):
    chunk = 4096 // _NTILES
    nsub = chunk // 32
    w = lax.axis_index("s") * 2 + lax.axis_index("c")
    t0 = w * chunk
    pltpu.sync_copy(i0.at[pl.ds(t0, chunk)], iv0)
    pltpu.sync_copy(i1.at[pl.ds(t0, chunk)], iv1)
    pltpu.sync_copy(r0.at[pl.ds(t0, chunk)], rv0)
    pltpu.sync_copy(r1.at[pl.ds(t0, chunk)], rv1)
    pltpu.sync_copy(ps, psv)
    for l in range(chunk // _L):
        sl = pl.ds(l * _L, _L)
        st0 = plsc.load_gather(psv, [iv0[sl]])
        st1 = plsc.load_gather(psv, [iv1[sl]])
        spv0 = st0 + rv0[sl]
        spv1 = st1 + rv1[sl]
        spf0[sl] = spv0
        spf1[sl] = spv1
        sp2d0[l // 2, pl.ds((l % 2) * _L, _L)] = spv0
        sp2d1[l // 2, pl.ds((l % 2) * _L, _L)] = spv1
    pltpu.sync_copy(spf0, sp0.at[pl.ds(t0, chunk)])
    pltpu.sync_copy(spf1, sp1.at[pl.ds(t0, chunk)])
    for j in range(nsub):
        pltpu.sync_copy(x2.at[pl.ds(t0 + j * 32, 32)], xbuf)
        pltpu.async_copy(xbuf, xg.at[sp2d0.at[j]], sem).wait()
        pltpu.async_copy(xbuf, xg.at[sp2d1.at[j]], sem).wait()


def _expert_body(be_ref, xg_ref, egT_ref, euT_ref, edT_ref, yg_ref):
    x = xg_ref[...]
    ge = jnp.dot(x, egT_ref[0], preferred_element_type=jnp.float32)
    ue = jnp.dot(x, euT_ref[0], preferred_element_type=jnp.float32)
    he = jax.nn.sigmoid(ge) * ue
    yg_ref[...] = jnp.dot(he, edT_ref[0], preferred_element_type=jnp.float32)


def _combine_body(xs, yg, sp0, sp1, w0, w1, out,
                  spv0, spv1, wv0, wv1, y0, y1, xsb, sem0, sem1):
    chunk = 4096 // _NTILES
    sub = 16
    H = 1024
    w = lax.axis_index("s") * 2 + lax.axis_index("c")
    t0 = w * chunk
    pltpu.sync_copy(sp0.at[pl.ds(t0, chunk)], spv0)
    pltpu.sync_copy(sp1.at[pl.ds(t0, chunk)], spv1)
    pltpu.sync_copy(w0.at[pl.ds(t0, chunk)], wv0.at[pl.ds(0, chunk)])
    pltpu.sync_copy(w1.at[pl.ds(t0, chunk)], wv1.at[pl.ds(0, chunk)])
    for j in range(chunk // sub):
        a0 = pltpu.async_copy(yg.at[spv0.at[pl.ds(j * sub, sub)]], y0, sem0)
        a1 = pltpu.async_copy(yg.at[spv1.at[pl.ds(j * sub, sub)]], y1, sem1)
        pltpu.sync_copy(xs.at[pl.ds(t0 + j * sub, sub)], xsb)
        a0.wait()
        a1.wait()

        def body(t, _):
            ws0 = wv0[pl.ds(j * sub + t, _L)][0]
            ws1 = wv1[pl.ds(j * sub + t, _L)][0]
            for l in range(H // _L):
                sl = pl.ds(l * _L, _L)
                xsb[t, sl] = xsb[t, sl] + ws0 * y0[t, sl] + ws1 * y1[t, sl]
            return 0

        lax.fori_loop(0, sub, body, 0)
        pltpu.sync_copy(xsb, out.at[pl.ds(t0 + j * sub, sub)])


def kernel(hidden_states, shared_gate_w, shared_up_w, shared_down_w,
           expert_gate_w, expert_up_w, expert_down_w, router_w):
    B, S, H = hidden_states.shape
    I = shared_gate_w.shape[0]
    T = B * S
    x2 = hidden_states.reshape(T, H)
    sgT = shared_gate_w.T
    suT = shared_up_w.T
    sdT = shared_down_w.T
    rT = router_w.T
    egT = expert_gate_w.transpose(0, 2, 1)
    euT = expert_up_w.transpose(0, 2, 1)
    edT = expert_down_w.transpose(0, 2, 1)

    nt = T // _TB
    nb = (T * _K) // _BLK + _E
    pp = nb * _BLK

    xs, meta, cnt = pl.pallas_call(
        _router_body,
        grid=(nt,),
        in_specs=[
            pl.BlockSpec((_TB, H), lambda t: (t, 0)),
            pl.BlockSpec((H, I), lambda t: (0, 0)),
            pl.BlockSpec((H, I), lambda t: (0, 0)),
            pl.BlockSpec((I, H), lambda t: (0, 0)),
            pl.BlockSpec((H, _E), lambda t: (0, 0)),
        ],
        out_specs=[
            pl.BlockSpec((_TB, H), lambda t: (t, 0)),
            pl.BlockSpec((1, _TB, 8), lambda t: (t, 0, 0)),
            pl.BlockSpec((8, 8), lambda t: (0, 0)),
        ],
        out_shape=[
            jax.ShapeDtypeStruct((T, H), jnp.float32),
            jax.ShapeDtypeStruct((nt, _TB, 8), jnp.float32),
            jax.ShapeDtypeStruct((8, 8), jnp.float32),
        ],
        scratch_shapes=[pltpu.VMEM((1, _E), jnp.float32)],
    )(x2, sgT, suT, sdT, rT)

    m2d = meta.reshape(T, 8)
    w0 = m2d[:, 0]
    w1 = m2d[:, 1]
    i0 = m2d[:, 2].astype(jnp.int32)
    i1 = m2d[:, 3].astype(jnp.int32)
    r0 = m2d[:, 4].astype(jnp.int32)
    r1 = m2d[:, 5].astype(jnp.int32)
    n = cnt[0].astype(jnp.int32)
    bc = (n + _BLK - 1) // _BLK
    cumbc = jnp.cumsum(bc)
    bstart = jnp.concatenate([jnp.zeros((1,), jnp.int32), cumbc[:-1]])
    pstart16 = jnp.pad(bstart * _BLK, (0, 16 - _E))
    be = jnp.clip(
        jnp.searchsorted(cumbc, jnp.arange(nb, dtype=jnp.int32), side="right"),
        0, _E - 1).astype(jnp.int32)

    mesh = plsc.VectorSubcoreMesh(core_axis_name="c", subcore_axis_name="s")
    chunk = T // _NTILES
    sc_params = pltpu.CompilerParams(needs_layout_passes=False)
    sc_scatter = pl.kernel(
        _scatter_body,
        compiler_params=sc_params,
        out_type=[
            jax.ShapeDtypeStruct((pp, H), jnp.float32),
            jax.ShapeDtypeStruct((T,), jnp.int32),
            jax.ShapeDtypeStruct((T,), jnp.int32),
        ],
        mesh=mesh,
        scratch_types=[
            pltpu.VMEM((chunk,), jnp.int32),
            pltpu.VMEM((chunk,), jnp.int32),
            pltpu.VMEM((chunk,), jnp.int32),
            pltpu.VMEM((chunk,), jnp.int32),
            pltpu.VMEM((16,), jnp.int32),
            pltpu.VMEM((chunk,), jnp.int32),
            pltpu.VMEM((chunk,), jnp.int32),
            pltpu.VMEM((chunk // 32, 32), jnp.int32),
            pltpu.VMEM((chunk // 32, 32), jnp.int32),
            pltpu.VMEM((32, H), jnp.float32),
            pltpu.SemaphoreType.DMA,
        ],
    )
    xg, sp0, sp1 = sc_scatter(x2, i0, i1, r0, r1, pstart16)

    yg = pl.pallas_call(
        _expert_body,
        grid_spec=pltpu.PrefetchScalarGridSpec(
            num_scalar_prefetch=1,
            grid=(nb,),
            in_specs=[
                pl.BlockSpec((_BLK, H), lambda b, be_r: (b, 0)),
                pl.BlockSpec((1, H, I), lambda b, be_r: (be_r[b], 0, 0)),
                pl.BlockSpec((1, H, I), lambda b, be_r: (be_r[b], 0, 0)),
                pl.BlockSpec((1, I, H), lambda b, be_r: (be_r[b], 0, 0)),
            ],
            out_specs=pl.BlockSpec((_BLK, H), lambda b, be_r: (b, 0)),
        ),
        out_shape=jax.ShapeDtypeStruct((pp, H), jnp.float32),
    )(be, xg, egT, euT, edT)

    sub = 16
    sc_combine = pl.kernel(
        _combine_body,
        compiler_params=sc_params,
        out_type=jax.ShapeDtypeStruct((T, H), jnp.float32),
        mesh=mesh,
        scratch_types=[
            pltpu.VMEM((chunk,), jnp.int32),
            pltpu.VMEM((chunk,), jnp.int32),
            pltpu.VMEM((chunk + _L,), jnp.float32),
            pltpu.VMEM((chunk + _L,), jnp.float32),
            pltpu.VMEM((sub, H), jnp.float32),
            pltpu.VMEM((sub, H), jnp.float32),
            pltpu.VMEM((sub, H), jnp.float32),
            pltpu.SemaphoreType.DMA,
            pltpu.SemaphoreType.DMA,
        ],
    )
    out2 = sc_combine(xs, yg, sp0, sp1, w0, w1)
    return out2.reshape(B, S, H)
